# baseline (speedup 1.0000x reference)
import math

import jax
import jax.numpy as jnp
from jax import lax
from jax.experimental import pallas as pl
from jax.experimental.pallas import tpu as pltpu
from jax.experimental.pallas import tpu_sc as plsc

N = 10000
E = 320000
MUL = 128
INV_SQRT_AVG = float(1.0 / math.sqrt(32.0))


def _norm_const():
    x = jnp.sqrt(2.0) * jax.scipy.special.erfinv(jnp.linspace(-1.0, 1.0, 100003)[1:-1])
    y = x * jax.nn.sigmoid(x)
    return float(jnp.sqrt(jnp.mean(y ** 2)))


C_SILU = _norm_const()

_NC = 2
_NS = 16
_NW = _NC * _NS
_GB = 128

_EH = E // 2
_HR = _EH // _GB
_GR0 = _HR // _NW
_GRX = _HR - _GR0 * _NW
_GIDX = 48
_QR = _HR // 2
_EQ = _QR * _GB
_SR0 = _QR // _NS
_SRX = _QR - _SR0 * _NS
_SIDX = 48
_RING = 3
_SRING = 2
_NPT = 624
_NTAIL = N - _NS * _NPT


def _mesh():
    return plsc.VectorSubcoreMesh(
        core_axis_name="c", subcore_axis_name="s",
        num_cores=_NC, num_subcores=_NS)


def _xw1_body(x_ref, w_ref, o_ref):
    o_ref[...] = jnp.dot(x_ref[...], w_ref[...],
                         preferred_element_type=jnp.float32)


def _xw1(x, W1):
    return pl.pallas_call(
        _xw1_body,
        out_shape=jax.ShapeDtypeStruct((N, 128), jnp.float32),
        grid=(5,),
        in_specs=[
            pl.BlockSpec((2000, 128), lambda i: (i, 0)),
            pl.BlockSpec((128, 128), lambda i: (0, 0)),
        ],
        out_specs=pl.BlockSpec((2000, 128), lambda i: (i, 0)),
    )(x, W1)


def _gather_body(tbl, idx2d_hbm, out_hbm, idx_v, b0, b1, b2, s0, s1, s2):
    c = lax.axis_index("c")
    s = lax.axis_index("s")
    wid = s * _NC + c
    nrows = jnp.where(wid < _GRX, _GR0 + 1, _GR0)
    rbase = _GR0 * wid + jnp.minimum(wid, _GRX)
    bufs = (b0, b1, b2)
    sems = (s0, s1, s2)
    rb8 = pl.multiple_of((rbase // 8) * 8, 8)
    off = rbase - rb8
    pltpu.sync_copy(idx2d_hbm.at[pl.ds(rb8, _GIDX)], idx_v)

    for b in range(_RING):
        @pl.when(b < nrows)
        def _prime(b=b):
            pltpu.async_copy(tbl.at[idx_v.at[off + b]], bufs[b], sems[b])

    ngroups = (_GR0 + 1 + _RING - 1) // _RING

    def group(g, carry):
        for b in range(_RING):
            j = g * _RING + b

            @pl.when(j < nrows)
            def _step(b=b, j=j):
                pltpu.make_async_copy(tbl.at[idx_v.at[0]], bufs[b], sems[b]).wait()
                pltpu.sync_copy(bufs[b],
                                out_hbm.at[pl.ds((rbase + j) * _GB, _GB)])
                nxt = j + _RING

                @pl.when(nxt < nrows)
                def _refill():
                    pltpu.async_copy(tbl.at[idx_v.at[off + nxt]], bufs[b], sems[b])
        return carry

    lax.fori_loop(0, ngroups, group, 0)


def _gather(xw1, senders2d):
    f = pl.kernel(
        _gather_body,
        out_type=jax.ShapeDtypeStruct((_EH, 128), jnp.float32),
        mesh=_mesh(),
        scratch_types=[
            pltpu.VMEM((_GIDX, _GB), jnp.int32),
            pltpu.VMEM((_GB, 128), jnp.float32),
            pltpu.VMEM((_GB, 128), jnp.float32),
            pltpu.VMEM((_GB, 128), jnp.float32),
            pltpu.SemaphoreType.DMA,
            pltpu.SemaphoreType.DMA,
            pltpu.SemaphoreType.DMA,
        ],
    )
    return f(xw1, senders2d)


def _edge_body(radial_ref, sh_ref, msg_ref, wr0_ref, br0_ref, wr1_ref, br1_ref,
               wr2_ref, o_ref):
    r = radial_ref[...].astype(jnp.bfloat16)
    h = jnp.dot(r, wr0_ref[...], preferred_element_type=jnp.float32) + br0_ref[...]
    h = h * jax.nn.sigmoid(h)
    h = jnp.dot(h.astype(jnp.bfloat16), wr1_ref[...],
                preferred_element_type=jnp.float32) + br1_ref[...]
    h = h * jax.nn.sigmoid(h)
    w01 = jnp.dot(h.astype(jnp.bfloat16), wr2_ref[...],
                  preferred_element_type=jnp.float32)
    m = msg_ref[...]
    t0 = m * w01[:, :MUL]
    t1 = m * w01[:, MUL:]
    o_ref[0] = t0 * sh_ref[:, 0:1]
    o_ref[1] = t1 * sh_ref[:, 1:2]
    o_ref[2] = t1 * sh_ref[:, 2:3]
    o_ref[3] = t1 * sh_ref[:, 3:4]


_EB = 2000


def _edge_planes(radial, sh, msg_h, Wr0, br0, Wr1, br1, Wr2c, quarter):
    g = _EQ // _EB
    off = quarter * g
    moff = (quarter % 2) * g

    return pl.pallas_call(
        _edge_body,
        out_shape=jax.ShapeDtypeStruct((4, _EQ, 128), jnp.float32),
        grid=(g,),
        in_specs=[
            pl.BlockSpec((_EB, 8), lambda i: (i + off, 0)),
            pl.BlockSpec((_EB, 9), lambda i: (i + off, 0)),
            pl.BlockSpec((_EB, 128), lambda i: (i + moff, 0)),
            pl.BlockSpec((8, 64), lambda i: (0, 0)),
            pl.BlockSpec((64,), lambda i: (0,)),
            pl.BlockSpec((64, 64), lambda i: (0, 0)),
            pl.BlockSpec((64,), lambda i: (0,)),
            pl.BlockSpec((64, 256), lambda i: (0, 0)),
        ],
        out_specs=pl.BlockSpec((4, _EB, 128), lambda i: (0, i, 0)),
    )(radial, sh, msg_h, Wr0, br0, Wr1, br1, Wr2c)


def _make_scatter_body(chained):
  def _scatter_body(mp_hbm, idx2d_hbm, init_hbm, out_hbm,
                    idx_v, b0, b1, acc_sh, s0, s1):
    c = lax.axis_index("c")
    s = lax.axis_index("s")
    nrows = jnp.where(s < _SRX, _SR0 + 1, _SR0)
    rbase = _SR0 * s + jnp.minimum(s, _SRX)
    bufs = (b0, b1)
    sems = (s0, s1)
    rb8 = pl.multiple_of((rbase // 8) * 8, 8)
    off = rbase - rb8
    pltpu.sync_copy(idx2d_hbm.at[pl.ds(rb8, _SIDX)], idx_v)

    for p in range(2):
        k = c * 2 + p
        init_at = ((lambda lo, n: init_hbm.at[k, pl.ds(lo, n)]) if chained
                   else (lambda lo, n: init_hbm.at[pl.ds(lo, n)]))
        pltpu.sync_copy(init_at(s * _NPT, _NPT),
                        acc_sh.at[pl.ds(s * _NPT, _NPT)])

        @pl.when(s == _NS - 1)
        def _zero_tail():
            pltpu.sync_copy(init_at(_NS * _NPT, _NTAIL),
                            acc_sh.at[pl.ds(_NS * _NPT, _NTAIL)])

        plsc.subcore_barrier()

        for b in range(_SRING):
            @pl.when(b < nrows)
            def _prime(b=b):
                pltpu.async_copy(
                    mp_hbm.at[k, pl.ds((rbase + b) * _GB, _GB)],
                    bufs[b], sems[b])

        ngroups = (_SR0 + 1 + _SRING - 1) // _SRING

        def group(g, carry):
            for b in range(_SRING):
                j = g * _SRING + b

                @pl.when(j < nrows)
                def _step(b=b, j=j):
                    pltpu.make_async_copy(
                        mp_hbm.at[k, pl.ds(rbase * _GB, _GB)],
                        bufs[b], sems[b]).wait()
                    pltpu.sync_copy(bufs[b], acc_sh.at[idx_v.at[off + j]],
                                    add=True)
                    nxt = j + _SRING

                    @pl.when(nxt < nrows)
                    def _refill():
                        pltpu.async_copy(
                            mp_hbm.at[k, pl.ds((rbase + nxt) * _GB, _GB)],
                            bufs[b], sems[b])
            return carry

        lax.fori_loop(0, ngroups, group, 0)
        plsc.subcore_barrier()
        pltpu.sync_copy(acc_sh.at[pl.ds(s * _NPT, _NPT)],
                        out_hbm.at[k, pl.ds(s * _NPT, _NPT)])

        @pl.when(s == _NS - 1)
        def _write_tail():
            pltpu.sync_copy(acc_sh.at[pl.ds(_NS * _NPT, _NTAIL)],
                            out_hbm.at[k, pl.ds(_NS * _NPT, _NTAIL)])


  return _scatter_body


def _scatter(mplanes, receivers2d, init, chained):
    f = pl.kernel(
        _make_scatter_body(chained),
        out_type=jax.ShapeDtypeStruct((4, N, 128), jnp.float32),
        mesh=_mesh(),
        scratch_types=[
            pltpu.VMEM((_SIDX, _GB), jnp.int32),
            pltpu.VMEM((_GB, 128), jnp.float32),
            pltpu.VMEM((_GB, 128), jnp.float32),
            pltpu.VMEM_SHARED((N, 128), jnp.float32),
            pltpu.SemaphoreType.DMA,
            pltpu.SemaphoreType.DMA,
        ],
    )
    return f(mplanes, receivers2d, init)


def _node_body(agg_ref, x_ref, wo0_ref, wsk_ref, wv_ref, sg_ref,
               o_ref):
    a0 = agg_ref[0] * INV_SQRT_AVG
    s = (jnp.dot(a0, wo0_ref[...], preferred_element_type=jnp.float32)
         + jnp.dot(x_ref[...], wsk_ref[...], preferred_element_type=jnp.float32))
    sc = s[:, :128]
    g = s[:, 128:]
    scal = sc * jax.nn.sigmoid(sc) * (1.0 / C_SILU)
    gates = g * jax.nn.sigmoid(g) * (1.0 / C_SILU)
    vmix = (jnp.dot(agg_ref[1], wv_ref[0], preferred_element_type=jnp.float32)
            + jnp.dot(agg_ref[2], wv_ref[1], preferred_element_type=jnp.float32)
            + jnp.dot(agg_ref[3], wv_ref[2], preferred_element_type=jnp.float32))
    vmix = vmix * INV_SQRT_AVG
    o_ref[:, :128] = scal
    o_ref[:, 128:] = jnp.dot(gates, sg_ref[...],
                             preferred_element_type=jnp.float32) * vmix


def _node_head(agg, x, W_out0, W_skip, Wv, Sg):
    NB = 2000
    g = N // NB
    return pl.pallas_call(
        _node_body,
        out_shape=jax.ShapeDtypeStruct((N, 320), jnp.float32),
        grid=(g,),
        in_specs=[
            pl.BlockSpec((4, NB, 128), lambda i: (0, i, 0)),
            pl.BlockSpec((NB, 128), lambda i: (i, 0)),
            pl.BlockSpec((128, 192), lambda i: (0, 0)),
            pl.BlockSpec((128, 192), lambda i: (0, 0)),
            pl.BlockSpec((3, 128, 192), lambda i: (0, 0, 0)),
            pl.BlockSpec((64, 192), lambda i: (0, 0)),
        ],
        out_specs=pl.BlockSpec((NB, 320), lambda i: (i, 0)),
    )(agg, x, W_out0, W_skip, Wv, Sg)


def kernel(x, sh, radial, senders, receivers, W1, Wr0, br0, Wr1, br1, Wr2,
           W_out0, W_out1, W_skip):
    senders = senders.astype(jnp.int32)
    receivers = receivers.astype(jnp.int32)
    Wr2c = Wr2[:, : 2 * MUL]
    eye = jnp.eye(64, dtype=jnp.float32)
    Sg = jnp.repeat(eye, 3, axis=1)
    col = jnp.arange(192, dtype=jnp.int32) % 3
    Wv = jnp.stack([W_out1 @ (Sg * (col == i)) for i in range(3)])
    zeros = jnp.zeros((N, 128), jnp.float32)

    s2d = senders.reshape(2, _HR, _GB)
    r2d = receivers.reshape(4, _QR, _GB)
    send_idx = [jnp.pad(s2d[h], ((0, 6), (0, 0))) for h in range(2)]
    recv_idx = [jnp.pad(r2d[q], ((0, 7), (0, 0))) for q in range(4)]

    Wr0b = Wr0.astype(jnp.bfloat16)
    Wr1b = Wr1.astype(jnp.bfloat16)
    Wr2b = Wr2c.astype(jnp.bfloat16)

    xw1 = _xw1(x, W1)
    msg0 = _gather(xw1, send_idx[0])
    msg1 = _gather(xw1, send_idx[1])
    msgs = (msg0, msg0, msg1, msg1)
    agg = zeros
    for q in range(4):
        mp = _edge_planes(radial, sh, msgs[q], Wr0b, br0, Wr1b, br1, Wr2b, q)
        agg = _scatter(mp, recv_idx[q], agg, chained=(q > 0))
    return _node_head(agg, x, W_out0, W_skip, Wv, Sg)


# speedup vs baseline: 1.0462x; 1.0462x over previous
import math

import jax
import jax.numpy as jnp
from jax import lax
from jax.experimental import pallas as pl
from jax.experimental.pallas import tpu as pltpu
from jax.experimental.pallas import tpu_sc as plsc

N = 10000
E = 320000
MUL = 128
INV_SQRT_AVG = float(1.0 / math.sqrt(32.0))


def _norm_const():
    x = jnp.sqrt(2.0) * jax.scipy.special.erfinv(jnp.linspace(-1.0, 1.0, 100003)[1:-1])
    y = x * jax.nn.sigmoid(x)
    return float(jnp.sqrt(jnp.mean(y ** 2)))


C_SILU = _norm_const()

_NC = 2
_NS = 16
_NW = _NC * _NS
_GB = 128

_EH = E // 2
_HR = _EH // _GB
_GR0 = _HR // _NW
_GRX = _HR - _GR0 * _NW
_GIDX = 48
_SR0 = _HR // _NS
_SRX = _HR - _SR0 * _NS
_SIDX = 88
_RING = 3
_SRING = 2
_NPT = 624
_NTAIL = N - _NS * _NPT


def _mesh():
    return plsc.VectorSubcoreMesh(
        core_axis_name="c", subcore_axis_name="s",
        num_cores=_NC, num_subcores=_NS)


def _xw1_body(x_ref, w_ref, o_ref):
    o_ref[...] = jnp.dot(x_ref[...], w_ref[...],
                         preferred_element_type=jnp.float32)


def _xw1(x, W1):
    return pl.pallas_call(
        _xw1_body,
        out_shape=jax.ShapeDtypeStruct((N, 128), jnp.float32),
        grid=(5,),
        in_specs=[
            pl.BlockSpec((2000, 128), lambda i: (i, 0)),
            pl.BlockSpec((128, 128), lambda i: (0, 0)),
        ],
        out_specs=pl.BlockSpec((2000, 128), lambda i: (i, 0)),
    )(x, W1)


def _gather_body(tbl, idx2d_hbm, out_hbm, idx_v, b0, b1, b2, s0, s1, s2):
    c = lax.axis_index("c")
    s = lax.axis_index("s")
    wid = s * _NC + c
    nrows = jnp.where(wid < _GRX, _GR0 + 1, _GR0)
    rbase = _GR0 * wid + jnp.minimum(wid, _GRX)
    bufs = (b0, b1, b2)
    sems = (s0, s1, s2)
    rb8 = pl.multiple_of((rbase // 8) * 8, 8)
    off = rbase - rb8
    pltpu.sync_copy(idx2d_hbm.at[pl.ds(rb8, _GIDX)], idx_v)

    for b in range(_RING):
        @pl.when(b < nrows)
        def _prime(b=b):
            pltpu.async_copy(tbl.at[idx_v.at[off + b]], bufs[b], sems[b])

    ngroups = (_GR0 + 1 + _RING - 1) // _RING

    def group(g, carry):
        for b in range(_RING):
            j = g * _RING + b

            @pl.when(j < nrows)
            def _step(b=b, j=j):
                pltpu.make_async_copy(tbl.at[idx_v.at[0]], bufs[b], sems[b]).wait()
                pltpu.sync_copy(bufs[b],
                                out_hbm.at[pl.ds((rbase + j) * _GB, _GB)])
                nxt = j + _RING

                @pl.when(nxt < nrows)
                def _refill():
                    pltpu.async_copy(tbl.at[idx_v.at[off + nxt]], bufs[b], sems[b])
        return carry

    lax.fori_loop(0, ngroups, group, 0)


def _gather(xw1, senders2d):
    f = pl.kernel(
        _gather_body,
        out_type=jax.ShapeDtypeStruct((_EH, 128), jnp.float32),
        mesh=_mesh(),
        scratch_types=[
            pltpu.VMEM((_GIDX, _GB), jnp.int32),
            pltpu.VMEM((_GB, 128), jnp.float32),
            pltpu.VMEM((_GB, 128), jnp.float32),
            pltpu.VMEM((_GB, 128), jnp.float32),
            pltpu.SemaphoreType.DMA,
            pltpu.SemaphoreType.DMA,
            pltpu.SemaphoreType.DMA,
        ],
    )
    return f(xw1, senders2d)


def _edge_body(radial_ref, sh_ref, msg_ref, wr0_ref, br0_ref, wr1_ref, br1_ref,
               wr2_ref, o_ref):
    r = radial_ref[...].astype(jnp.bfloat16)
    h = jnp.dot(r, wr0_ref[...], preferred_element_type=jnp.float32) + br0_ref[...]
    h = h * jax.nn.sigmoid(h)
    h = jnp.dot(h.astype(jnp.bfloat16), wr1_ref[...],
                preferred_element_type=jnp.float32) + br1_ref[...]
    h = h * jax.nn.sigmoid(h)
    w01 = jnp.dot(h.astype(jnp.bfloat16), wr2_ref[...],
                  preferred_element_type=jnp.float32)
    m = msg_ref[...]
    t0 = m * w01[:, :MUL]
    t1 = m * w01[:, MUL:]
    o_ref[0] = t0 * sh_ref[:, 0:1]
    o_ref[1] = t1 * sh_ref[:, 1:2]
    o_ref[2] = t1 * sh_ref[:, 2:3]
    o_ref[3] = t1 * sh_ref[:, 3:4]


_EB = 4000


def _edge_planes(radial, sh, msg_h, Wr0, br0, Wr1, br1, Wr2c, half):
    g = _EH // _EB
    off = half * g

    return pl.pallas_call(
        _edge_body,
        out_shape=jax.ShapeDtypeStruct((4, _EH, 128), jnp.float32),
        grid=(g,),
        in_specs=[
            pl.BlockSpec((_EB, 8), lambda i: (i + off, 0)),
            pl.BlockSpec((_EB, 9), lambda i: (i + off, 0)),
            pl.BlockSpec((_EB, 128), lambda i: (i, 0)),
            pl.BlockSpec((8, 64), lambda i: (0, 0)),
            pl.BlockSpec((64,), lambda i: (0,)),
            pl.BlockSpec((64, 64), lambda i: (0, 0)),
            pl.BlockSpec((64,), lambda i: (0,)),
            pl.BlockSpec((64, 256), lambda i: (0, 0)),
        ],
        out_specs=pl.BlockSpec((4, _EB, 128), lambda i: (0, i, 0)),
    )(radial, sh, msg_h, Wr0, br0, Wr1, br1, Wr2c)


def _make_scatter_body(chained):
  def _scatter_body(mp_hbm, idx2d_hbm, init_hbm, out_hbm,
                    idx_v, b0, b1, acc_sh, s0, s1):
    c = lax.axis_index("c")
    s = lax.axis_index("s")
    nrows = jnp.where(s < _SRX, _SR0 + 1, _SR0)
    rbase = _SR0 * s + jnp.minimum(s, _SRX)
    bufs = (b0, b1)
    sems = (s0, s1)
    rb8 = pl.multiple_of((rbase // 8) * 8, 8)
    off = rbase - rb8
    pltpu.sync_copy(idx2d_hbm.at[pl.ds(rb8, _SIDX)], idx_v)

    for p in range(2):
        k = c * 2 + p
        init_at = ((lambda lo, n: init_hbm.at[k, pl.ds(lo, n)]) if chained
                   else (lambda lo, n: init_hbm.at[pl.ds(lo, n)]))
        pltpu.sync_copy(init_at(s * _NPT, _NPT),
                        acc_sh.at[pl.ds(s * _NPT, _NPT)])

        @pl.when(s == _NS - 1)
        def _zero_tail():
            pltpu.sync_copy(init_at(_NS * _NPT, _NTAIL),
                            acc_sh.at[pl.ds(_NS * _NPT, _NTAIL)])

        plsc.subcore_barrier()

        for b in range(_SRING):
            @pl.when(b < nrows)
            def _prime(b=b):
                pltpu.async_copy(
                    mp_hbm.at[k, pl.ds((rbase + b) * _GB, _GB)],
                    bufs[b], sems[b])

        ngroups = (_SR0 + 1 + _SRING - 1) // _SRING

        def group(g, carry):
            for b in range(_SRING):
                j = g * _SRING + b

                @pl.when(j < nrows)
                def _step(b=b, j=j):
                    pltpu.make_async_copy(
                        mp_hbm.at[k, pl.ds(rbase * _GB, _GB)],
                        bufs[b], sems[b]).wait()
                    pltpu.sync_copy(bufs[b], acc_sh.at[idx_v.at[off + j]],
                                    add=True)
                    nxt = j + _SRING

                    @pl.when(nxt < nrows)
                    def _refill():
                        pltpu.async_copy(
                            mp_hbm.at[k, pl.ds((rbase + nxt) * _GB, _GB)],
                            bufs[b], sems[b])
            return carry

        lax.fori_loop(0, ngroups, group, 0)
        plsc.subcore_barrier()
        pltpu.sync_copy(acc_sh.at[pl.ds(s * _NPT, _NPT)],
                        out_hbm.at[k, pl.ds(s * _NPT, _NPT)])

        @pl.when(s == _NS - 1)
        def _write_tail():
            pltpu.sync_copy(acc_sh.at[pl.ds(_NS * _NPT, _NTAIL)],
                            out_hbm.at[k, pl.ds(_NS * _NPT, _NTAIL)])


  return _scatter_body


def _scatter(mplanes, receivers2d, init, chained):
    f = pl.kernel(
        _make_scatter_body(chained),
        out_type=jax.ShapeDtypeStruct((4, N, 128), jnp.float32),
        mesh=_mesh(),
        scratch_types=[
            pltpu.VMEM((_SIDX, _GB), jnp.int32),
            pltpu.VMEM((_GB, 128), jnp.float32),
            pltpu.VMEM((_GB, 128), jnp.float32),
            pltpu.VMEM_SHARED((N, 128), jnp.float32),
            pltpu.SemaphoreType.DMA,
            pltpu.SemaphoreType.DMA,
        ],
    )
    return f(mplanes, receivers2d, init)


def _node_body(agg_ref, x_ref, wo0_ref, wsk_ref, wv_ref, sg_ref,
               o_ref):
    a0 = agg_ref[0] * INV_SQRT_AVG
    s = (jnp.dot(a0, wo0_ref[...], preferred_element_type=jnp.float32)
         + jnp.dot(x_ref[...], wsk_ref[...], preferred_element_type=jnp.float32))
    sc = s[:, :128]
    g = s[:, 128:]
    scal = sc * jax.nn.sigmoid(sc) * (1.0 / C_SILU)
    gates = g * jax.nn.sigmoid(g) * (1.0 / C_SILU)
    vmix = (jnp.dot(agg_ref[1], wv_ref[0], preferred_element_type=jnp.float32)
            + jnp.dot(agg_ref[2], wv_ref[1], preferred_element_type=jnp.float32)
            + jnp.dot(agg_ref[3], wv_ref[2], preferred_element_type=jnp.float32))
    vmix = vmix * INV_SQRT_AVG
    o_ref[:, :128] = scal
    o_ref[:, 128:] = jnp.dot(gates, sg_ref[...],
                             preferred_element_type=jnp.float32) * vmix


def _node_head(agg, x, W_out0, W_skip, Wv, Sg):
    NB = 2000
    g = N // NB
    return pl.pallas_call(
        _node_body,
        out_shape=jax.ShapeDtypeStruct((N, 320), jnp.float32),
        grid=(g,),
        in_specs=[
            pl.BlockSpec((4, NB, 128), lambda i: (0, i, 0)),
            pl.BlockSpec((NB, 128), lambda i: (i, 0)),
            pl.BlockSpec((128, 192), lambda i: (0, 0)),
            pl.BlockSpec((128, 192), lambda i: (0, 0)),
            pl.BlockSpec((3, 128, 192), lambda i: (0, 0, 0)),
            pl.BlockSpec((64, 192), lambda i: (0, 0)),
        ],
        out_specs=pl.BlockSpec((NB, 320), lambda i: (i, 0)),
    )(agg, x, W_out0, W_skip, Wv, Sg)


def kernel(x, sh, radial, senders, receivers, W1, Wr0, br0, Wr1, br1, Wr2,
           W_out0, W_out1, W_skip):
    senders = senders.astype(jnp.int32)
    receivers = receivers.astype(jnp.int32)
    Wr2c = Wr2[:, : 2 * MUL]
    eye = jnp.eye(64, dtype=jnp.float32)
    Sg = jnp.repeat(eye, 3, axis=1)
    col = jnp.arange(192, dtype=jnp.int32) % 3
    Wv = jnp.stack([W_out1 @ (Sg * (col == i)) for i in range(3)])
    zeros = jnp.zeros((N, 128), jnp.float32)

    s2d = senders.reshape(2, _HR, _GB)
    r2d = receivers.reshape(2, _HR, _GB)
    send_idx = [jnp.pad(s2d[h], ((0, 6), (0, 0))) for h in range(2)]
    recv_idx = [jnp.pad(r2d[h], ((0, 6), (0, 0))) for h in range(2)]

    Wr0b = Wr0.astype(jnp.bfloat16)
    Wr1b = Wr1.astype(jnp.bfloat16)
    Wr2b = Wr2c.astype(jnp.bfloat16)

    xw1 = _xw1(x, W1)
    msg0 = _gather(xw1, send_idx[0])
    msg1 = _gather(xw1, send_idx[1])
    mp0 = _edge_planes(radial, sh, msg0, Wr0b, br0, Wr1b, br1, Wr2b, 0)
    mp1 = _edge_planes(radial, sh, msg1, Wr0b, br0, Wr1b, br1, Wr2b, 1)
    agg0 = _scatter(mp0, recv_idx[0], zeros, chained=False)
    agg1 = _scatter(mp1, recv_idx[1], agg0, chained=True)
    return _node_head(agg1, x, W_out0, W_skip, Wv, Sg)
